# Initial kernel scaffold; baseline (speedup 1.0000x reference)
import jax
import jax.numpy as jnp
from jax.experimental import pallas as pl


def kernel(x, Wc, bc, gamma, beta):
    raise NotImplementedError("write your pallas kernel here")


# speedup vs baseline: 30.5470x; 30.5470x over previous
import numpy as np
import jax
import jax.numpy as jnp
from jax.experimental import pallas as pl

_B, _NODE, _C = 4, 32, 256
_N = _NODE * _NODE
_K1 = 11
_KS, _STRIDE = 5, 2
_EPS = 1e-5
_BR = 128
_NB = _N // _BR


def _local_parts():
    size, ks, stride = _NODE, _KS, _STRIDE
    inp = np.arange(size * size).reshape(size, size)
    patches = []
    for i in range(0, size - ks + 1, stride):
        for j in range(0, size - ks + 1, stride):
            patches.append(inp[i:i + ks, j:j + ks].reshape(-1))
    inp_unf = np.stack(patches, axis=0)
    edge = inp_unf.shape[0]
    H = np.zeros((size * size, edge), dtype=np.float32)
    for e in range(edge):
        H[inp_unf[e], e] = 1.0
    L = (H @ H.T) / float(ks * ks)
    cloc = H.sum(axis=1)
    return L.astype(np.float32), cloc.astype(np.float32)


_LOC_L, _LOC_CLOC = _local_parts()


def _row_kth_members(s, kvec):
    r, n = s.shape
    neg = (s < 0).astype(jnp.int32)
    c0 = jnp.sum(neg, axis=1, keepdims=True)
    cond = kvec <= c0
    prefix = jnp.where(cond, jnp.int32(-2147483648), jnp.int32(0))
    kk = jnp.where(cond, kvec, kvec - c0)
    for b in range(30, -1, -1):
        eq = (s >> (b + 1)) == (prefix >> (b + 1))
        bit0 = (s & (1 << b)) == 0
        c0 = jnp.sum((eq & bit0).astype(jnp.int32), axis=1, keepdims=True)
        cond = kk <= c0
        prefix = jnp.where(cond, prefix, prefix | (1 << b))
        kk = jnp.where(cond, kk, kk - c0)
    v = prefix
    lt = s < v
    c_lt = jnp.sum(lt.astype(jnp.int32), axis=1, keepdims=True)
    eqm = (s == v).astype(jnp.int32)
    csum = eqm
    d = 1
    while d < n:
        shifted = jnp.concatenate(
            [jnp.zeros((r, d), jnp.int32), csum[:, : n - d]], axis=1)
        csum = csum + shifted
        d *= 2
    excl = csum - eqm
    tie_take = eqm.astype(jnp.bool_) & (excl < (kvec - c_lt))
    return (lt | tie_take).astype(jnp.float32)


def _dist_body(xb_ref, xf_ref, s_ref):
    xb = xb_ref[0]
    xf = xf_ref[0]
    xx = jax.lax.dot_general(xb, xf, (((1,), (1,)), ((), ())),
                             preferred_element_type=jnp.float32)
    xsqb = jnp.sum(xb * xb, axis=1, keepdims=True)
    xsqf = jnp.reshape(jnp.sum(xf * xf, axis=1, keepdims=True), (1, _N))
    D = xsqb + (-2.0 * xx) + xsqf
    bits = jax.lax.bitcast_convert_type(D, jnp.int32)
    s_ref[0] = jnp.where(bits < 0, bits ^ jnp.int32(0x7FFFFFFF), bits)


def _deg_body(s_ref, dv_ref):
    m1 = _row_kth_members(s_ref[0], jnp.full((_BR, 1), _K1, jnp.int32))
    part = jnp.sum(m1, axis=0, keepdims=True)

    @pl.when(pl.program_id(1) == 0)
    def _init():
        dv_ref[0] = part

    @pl.when(pl.program_id(1) != 0)
    def _acc():
        dv_ref[0] += part


def _inc_body(s_ref, dvc_ref, af_ref, de_ref, dvt_ref):
    kvec = dvc_ref[0].astype(jnp.int32)
    m2 = _row_kth_members(s_ref[0], kvec)
    ri = jax.lax.broadcasted_iota(jnp.int32, (_BR, _N), 0) \
        + pl.program_id(1) * _BR
    ci = jax.lax.broadcasted_iota(jnp.int32, (_BR, _N), 1)
    af = jnp.where(ri == ci, 1.0, m2)
    af_ref[0] = af
    de_ref[0] = jnp.sum(af, axis=1, keepdims=True)
    part = jnp.sum(af, axis=0, keepdims=True)

    @pl.when(pl.program_id(1) == 0)
    def _init():
        dvt_ref[0] = part

    @pl.when(pl.program_id(1) != 0)
    def _acc():
        dvt_ref[0] += part


def _h_body(x_ref, Wc_ref, bc_ref, h_ref):
    h = jax.lax.dot_general(x_ref[0], Wc_ref[...], (((1,), (1,)), ((), ())),
                            preferred_element_type=jnp.float32)
    h_ref[0] = h + bc_ref[...]


def _g_body(afc_ref, aff_ref, de_ref, dvtr_ref, dvtc_ref, clr_ref, clc_ref,
            L_ref, h_ref, P_ref):
    w = 1.0 / de_ref[0]
    aw = aff_ref[0] * w
    S = jax.lax.dot_general(afc_ref[0], aw, (((0,), (0,)), ((), ())),
                            preferred_element_type=jnp.float32)
    sc = jax.lax.rsqrt(dvtc_ref[0] + clc_ref[...])
    sr = jax.lax.rsqrt(dvtr_ref[0] + clr_ref[...])
    G = (S + L_ref[...]) * sc * sr
    P_ref[0] = jax.lax.dot_general(G, h_ref[0], (((1,), (0,)), ((), ())),
                                   preferred_element_type=jnp.float32)


def _bn_body(P_ref, x_ref, gamma_ref, beta_ref, o_ref):
    P = jnp.reshape(P_ref[...], (_B * _N, _C))
    m = jnp.mean(P, axis=0, keepdims=True)
    d = P - m
    var = jnp.mean(d * d, axis=0, keepdims=True)
    hn = d / jnp.sqrt(var + _EPS) * gamma_ref[...] + beta_ref[...]
    hr = jnp.maximum(hn, 0.0)
    o_ref[...] = jnp.reshape(hr, (_B, _N, _C)) + x_ref[...]


@jax.jit
def kernel(x, Wc, bc, gamma, beta):
    L = jnp.asarray(_LOC_L)
    clc = jnp.asarray(_LOC_CLOC).reshape(_N, 1)
    clr = jnp.asarray(_LOC_CLOC).reshape(1, _N)

    s = pl.pallas_call(
        _dist_body,
        grid=(_B, _NB),
        in_specs=[
            pl.BlockSpec((1, _BR, _C), lambda b, j: (b, j, 0)),
            pl.BlockSpec((1, _N, _C), lambda b, j: (b, 0, 0)),
        ],
        out_specs=pl.BlockSpec((1, _BR, _N), lambda b, j: (b, j, 0)),
        out_shape=jax.ShapeDtypeStruct((_B, _N, _N), jnp.int32),
    )(x, x)

    dv_row = pl.pallas_call(
        _deg_body,
        grid=(_B, _NB),
        in_specs=[pl.BlockSpec((1, _BR, _N), lambda b, j: (b, j, 0))],
        out_specs=pl.BlockSpec((1, 1, _N), lambda b, j: (b, 0, 0)),
        out_shape=jax.ShapeDtypeStruct((_B, 1, _N), jnp.float32),
    )(s)
    dv_col = jnp.swapaxes(dv_row, 1, 2)

    af, de, dvt_row = pl.pallas_call(
        _inc_body,
        grid=(_B, _NB),
        in_specs=[
            pl.BlockSpec((1, _BR, _N), lambda b, j: (b, j, 0)),
            pl.BlockSpec((1, _BR, 1), lambda b, j: (b, j, 0)),
        ],
        out_specs=[
            pl.BlockSpec((1, _BR, _N), lambda b, j: (b, j, 0)),
            pl.BlockSpec((1, _BR, 1), lambda b, j: (b, j, 0)),
            pl.BlockSpec((1, 1, _N), lambda b, j: (b, 0, 0)),
        ],
        out_shape=[
            jax.ShapeDtypeStruct((_B, _N, _N), jnp.float32),
            jax.ShapeDtypeStruct((_B, _N, 1), jnp.float32),
            jax.ShapeDtypeStruct((_B, 1, _N), jnp.float32),
        ],
    )(s, dv_col)
    dvt_col = jnp.swapaxes(dvt_row, 1, 2)

    h = pl.pallas_call(
        _h_body,
        grid=(_B,),
        in_specs=[
            pl.BlockSpec((1, _N, _C), lambda b: (b, 0, 0)),
            pl.BlockSpec((_C, _C), lambda b: (0, 0)),
            pl.BlockSpec((1, _C), lambda b: (0, 0)),
        ],
        out_specs=pl.BlockSpec((1, _N, _C), lambda b: (b, 0, 0)),
        out_shape=jax.ShapeDtypeStruct((_B, _N, _C), jnp.float32),
    )(x, Wc, bc.reshape(1, _C))

    P = pl.pallas_call(
        _g_body,
        grid=(_B, _NB),
        in_specs=[
            pl.BlockSpec((1, _N, _BR), lambda b, j: (b, 0, j)),
            pl.BlockSpec((1, _N, _N), lambda b, j: (b, 0, 0)),
            pl.BlockSpec((1, _N, 1), lambda b, j: (b, 0, 0)),
            pl.BlockSpec((1, 1, _N), lambda b, j: (b, 0, 0)),
            pl.BlockSpec((1, _BR, 1), lambda b, j: (b, j, 0)),
            pl.BlockSpec((1, _N), lambda b, j: (0, 0)),
            pl.BlockSpec((_BR, 1), lambda b, j: (j, 0)),
            pl.BlockSpec((_BR, _N), lambda b, j: (j, 0)),
            pl.BlockSpec((1, _N, _C), lambda b, j: (b, 0, 0)),
        ],
        out_specs=pl.BlockSpec((1, _BR, _C), lambda b, j: (b, j, 0)),
        out_shape=jax.ShapeDtypeStruct((_B, _N, _C), jnp.float32),
    )(af, af, de, dvt_row, dvt_col, clr, clc, L, h)

    out = pl.pallas_call(
        _bn_body,
        out_shape=jax.ShapeDtypeStruct((_B, _N, _C), jnp.float32),
    )(P, x, gamma.reshape(1, _C), beta.reshape(1, _C))
    return out


# speedup vs baseline: 38.0381x; 1.2452x over previous
import numpy as np
import jax
import jax.numpy as jnp
from jax.experimental import pallas as pl

_B, _NODE, _C = 4, 32, 256
_N = _NODE * _NODE
_K1 = 11
_KS, _STRIDE = 5, 2
_EPS = 1e-5
_BR = 128
_NB = _N // _BR


def _local_parts():
    size, ks, stride = _NODE, _KS, _STRIDE
    inp = np.arange(size * size).reshape(size, size)
    patches = []
    for i in range(0, size - ks + 1, stride):
        for j in range(0, size - ks + 1, stride):
            patches.append(inp[i:i + ks, j:j + ks].reshape(-1))
    inp_unf = np.stack(patches, axis=0)
    edge = inp_unf.shape[0]
    H = np.zeros((size * size, edge), dtype=np.float32)
    for e in range(edge):
        H[inp_unf[e], e] = 1.0
    L = (H @ H.T) / float(ks * ks)
    cloc = H.sum(axis=1)
    return L.astype(np.float32), cloc.astype(np.float32)


_LOC_L, _LOC_CLOC = _local_parts()


def _col_kth_members(st, kvec):
    n, p = st.shape
    neg = (st < 0).astype(jnp.int32)
    c0 = jnp.sum(neg, axis=0, keepdims=True)
    cond = kvec <= c0
    prefix = jnp.where(cond, jnp.int32(-2147483648), jnp.int32(0))
    kk = jnp.where(cond, kvec, kvec - c0)
    for b in range(30, -1, -1):
        eq = (st >> (b + 1)) == (prefix >> (b + 1))
        bit0 = (st & (1 << b)) == 0
        c0 = jnp.sum((eq & bit0).astype(jnp.int32), axis=0, keepdims=True)
        cond = kk <= c0
        prefix = jnp.where(cond, prefix, prefix | (1 << b))
        kk = jnp.where(cond, kk, kk - c0)
    v = prefix
    lt = st < v
    c_lt = jnp.sum(lt.astype(jnp.int32), axis=0, keepdims=True)
    eqm = (st == v).astype(jnp.int32)
    csum = eqm
    d = 1
    while d < n:
        shifted = jnp.concatenate(
            [jnp.zeros((d, p), jnp.int32), csum[: n - d, :]], axis=0)
        csum = csum + shifted
        d *= 2
    excl = csum - eqm
    tie_take = eqm.astype(jnp.bool_) & (excl < (kvec - c_lt))
    return (lt | tie_take).astype(jnp.float32)


def _dist_body(xb_ref, xf_ref, st_ref):
    xb = xb_ref[0]
    xf = xf_ref[0]
    xxt = jax.lax.dot_general(xf, xb, (((1,), (1,)), ((), ())),
                              preferred_element_type=jnp.float32)
    xsqp = jax.lax.dot_general(jnp.ones((1, _C), jnp.float32), xb * xb,
                               (((1,), (1,)), ((), ())),
                               preferred_element_type=jnp.float32)
    xsqn = jnp.sum(xf * xf, axis=1, keepdims=True)
    D = xsqp + (-2.0 * xxt) + xsqn
    bits = jax.lax.bitcast_convert_type(D, jnp.int32)
    st_ref[0] = jnp.where(bits < 0, bits ^ jnp.int32(0x7FFFFFFF), bits)


def _deg_body(st_ref, dv_ref):
    m1 = _col_kth_members(st_ref[0], jnp.full((1, _BR), _K1, jnp.int32))
    part = jnp.sum(m1, axis=1, keepdims=True)

    @pl.when(pl.program_id(1) == 0)
    def _init():
        dv_ref[0] = part

    @pl.when(pl.program_id(1) != 0)
    def _acc():
        dv_ref[0] += part


def _inc_body(st_ref, dvr_ref, nh_ref, de_ref, dvt_ref):
    kvec = dvr_ref[0].astype(jnp.int32)
    m2 = _col_kth_members(st_ref[0], kvec)
    ri = jax.lax.broadcasted_iota(jnp.int32, (_N, _BR), 0)
    ci = jax.lax.broadcasted_iota(jnp.int32, (_N, _BR), 1) \
        + pl.program_id(1) * _BR
    nh = jnp.where(ri == ci, 1.0, m2)
    nh_ref[0] = nh
    de_ref[0] = jnp.sum(nh, axis=0, keepdims=True)
    part = jnp.sum(nh, axis=1, keepdims=True)

    @pl.when(pl.program_id(1) == 0)
    def _init():
        dvt_ref[0] = part

    @pl.when(pl.program_id(1) != 0)
    def _acc():
        dvt_ref[0] += part


def _h_body(x_ref, Wc_ref, bc_ref, h_ref):
    h = jax.lax.dot_general(x_ref[0], Wc_ref[...], (((1,), (1,)), ((), ())),
                            preferred_element_type=jnp.float32)
    h_ref[0] = h + bc_ref[...]


def _g_body(nhb_ref, nhf_ref, der_ref, dvtr_ref, dvtc_ref, clr_ref, clc_ref,
            L_ref, h_ref, P_ref):
    w = 1.0 / der_ref[0]
    aw = nhb_ref[0] * w
    S = jax.lax.dot_general(aw, nhf_ref[0], (((1,), (1,)), ((), ())),
                            preferred_element_type=jnp.float32)
    sc = jax.lax.rsqrt(dvtc_ref[0] + clc_ref[...])
    sr = jax.lax.rsqrt(dvtr_ref[0] + clr_ref[...])
    G = (S + L_ref[...]) * sc * sr
    P_ref[0] = jax.lax.dot_general(G, h_ref[0], (((1,), (0,)), ((), ())),
                                   preferred_element_type=jnp.float32)


def _bn_body(P_ref, x_ref, gamma_ref, beta_ref, o_ref):
    P = jnp.reshape(P_ref[...], (_B * _N, _C))
    m = jnp.mean(P, axis=0, keepdims=True)
    d = P - m
    var = jnp.mean(d * d, axis=0, keepdims=True)
    hn = d / jnp.sqrt(var + _EPS) * gamma_ref[...] + beta_ref[...]
    hr = jnp.maximum(hn, 0.0)
    o_ref[...] = jnp.reshape(hr, (_B, _N, _C)) + x_ref[...]


@jax.jit
def kernel(x, Wc, bc, gamma, beta):
    L = jnp.asarray(_LOC_L)
    clc = jnp.asarray(_LOC_CLOC).reshape(_N, 1)
    clr = jnp.asarray(_LOC_CLOC).reshape(1, _N)

    st = pl.pallas_call(
        _dist_body,
        grid=(_B, _NB),
        in_specs=[
            pl.BlockSpec((1, _BR, _C), lambda b, j: (b, j, 0)),
            pl.BlockSpec((1, _N, _C), lambda b, j: (b, 0, 0)),
        ],
        out_specs=pl.BlockSpec((1, _N, _BR), lambda b, j: (b, 0, j)),
        out_shape=jax.ShapeDtypeStruct((_B, _N, _N), jnp.int32),
    )(x, x)

    dv_col = pl.pallas_call(
        _deg_body,
        grid=(_B, _NB),
        in_specs=[pl.BlockSpec((1, _N, _BR), lambda b, j: (b, 0, j))],
        out_specs=pl.BlockSpec((1, _N, 1), lambda b, j: (b, 0, 0)),
        out_shape=jax.ShapeDtypeStruct((_B, _N, 1), jnp.float32),
    )(st)
    dv_row = jnp.swapaxes(dv_col, 1, 2)

    nh, de_row, dvt_col = pl.pallas_call(
        _inc_body,
        grid=(_B, _NB),
        in_specs=[
            pl.BlockSpec((1, _N, _BR), lambda b, j: (b, 0, j)),
            pl.BlockSpec((1, 1, _BR), lambda b, j: (b, 0, j)),
        ],
        out_specs=[
            pl.BlockSpec((1, _N, _BR), lambda b, j: (b, 0, j)),
            pl.BlockSpec((1, 1, _BR), lambda b, j: (b, 0, j)),
            pl.BlockSpec((1, _N, 1), lambda b, j: (b, 0, 0)),
        ],
        out_shape=[
            jax.ShapeDtypeStruct((_B, _N, _N), jnp.float32),
            jax.ShapeDtypeStruct((_B, 1, _N), jnp.float32),
            jax.ShapeDtypeStruct((_B, _N, 1), jnp.float32),
        ],
    )(st, dv_row)
    dvt_row = jnp.swapaxes(dvt_col, 1, 2)

    h = pl.pallas_call(
        _h_body,
        grid=(_B,),
        in_specs=[
            pl.BlockSpec((1, _N, _C), lambda b: (b, 0, 0)),
            pl.BlockSpec((_C, _C), lambda b: (0, 0)),
            pl.BlockSpec((1, _C), lambda b: (0, 0)),
        ],
        out_specs=pl.BlockSpec((1, _N, _C), lambda b: (b, 0, 0)),
        out_shape=jax.ShapeDtypeStruct((_B, _N, _C), jnp.float32),
    )(x, Wc, bc.reshape(1, _C))

    P = pl.pallas_call(
        _g_body,
        grid=(_B, _NB),
        in_specs=[
            pl.BlockSpec((1, _BR, _N), lambda b, j: (b, j, 0)),
            pl.BlockSpec((1, _N, _N), lambda b, j: (b, 0, 0)),
            pl.BlockSpec((1, 1, _N), lambda b, j: (b, 0, 0)),
            pl.BlockSpec((1, 1, _N), lambda b, j: (b, 0, 0)),
            pl.BlockSpec((1, _BR, 1), lambda b, j: (b, j, 0)),
            pl.BlockSpec((1, _N), lambda b, j: (0, 0)),
            pl.BlockSpec((_BR, 1), lambda b, j: (j, 0)),
            pl.BlockSpec((_BR, _N), lambda b, j: (j, 0)),
            pl.BlockSpec((1, _N, _C), lambda b, j: (b, 0, 0)),
        ],
        out_specs=pl.BlockSpec((1, _BR, _C), lambda b, j: (b, j, 0)),
        out_shape=jax.ShapeDtypeStruct((_B, _N, _C), jnp.float32),
    )(nh, nh, de_row, dvt_row, dvt_col, clr, clc, L, h)

    out = pl.pallas_call(
        _bn_body,
        out_shape=jax.ShapeDtypeStruct((_B, _N, _C), jnp.float32),
    )(P, x, gamma.reshape(1, _C), beta.reshape(1, _C))
    return out


# speedup vs baseline: 50.0764x; 1.3165x over previous
import numpy as np
import jax
import jax.numpy as jnp
from jax.experimental import pallas as pl

_B, _NODE, _C = 4, 32, 256
_N = _NODE * _NODE
_K1 = 11
_KS, _STRIDE = 5, 2
_EPS = 1e-5
_BR = 128
_NB = _N // _BR


def _local_parts():
    size, ks, stride = _NODE, _KS, _STRIDE
    inp = np.arange(size * size).reshape(size, size)
    patches = []
    for i in range(0, size - ks + 1, stride):
        for j in range(0, size - ks + 1, stride):
            patches.append(inp[i:i + ks, j:j + ks].reshape(-1))
    inp_unf = np.stack(patches, axis=0)
    edge = inp_unf.shape[0]
    H = np.zeros((size * size, edge), dtype=np.float32)
    for e in range(edge):
        H[inp_unf[e], e] = 1.0
    L = (H @ H.T) / float(ks * ks)
    cloc = H.sum(axis=1)
    return L.astype(np.float32), cloc.astype(np.float32)


_LOC_L, _LOC_CLOC = _local_parts()


def _members_from_threshold(st, v, kvec):
    n, p = st.shape
    lt = st < v
    c_lt = jnp.sum(lt.astype(jnp.int32), axis=0, keepdims=True)
    eqm = (st == v).astype(jnp.int32)
    csum = eqm
    d = 1
    while d < n:
        shifted = jnp.concatenate(
            [jnp.zeros((d, p), jnp.int32), csum[: n - d, :]], axis=0)
        csum = csum + shifted
        d *= 2
    excl = csum - eqm
    tie_take = eqm.astype(jnp.bool_) & (excl < (kvec - c_lt))
    return lt | tie_take


def _threshold_extract(st, k):
    rem = st
    removed = jnp.zeros((1, st.shape[1]), jnp.int32)
    v = jnp.full((1, st.shape[1]), -2147483648, jnp.int32)
    for _ in range(k):
        cur = jnp.min(rem, axis=0, keepdims=True)
        isv = rem == cur
        ccur = jnp.sum(isv.astype(jnp.int32), axis=0, keepdims=True)
        upd = removed < k
        v = jnp.where(upd, cur, v)
        removed = removed + jnp.where(upd, ccur, 0)
        rem = jnp.where(isv, jnp.int32(2147483647), rem)
    return v


def _threshold_radix(st, kvec):
    neg = (st < 0).astype(jnp.int32)
    c0 = jnp.sum(neg, axis=0, keepdims=True)
    cond = kvec <= c0
    prefix = jnp.where(cond, jnp.int32(-2147483648), jnp.int32(0))
    kk = jnp.where(cond, kvec, kvec - c0)
    for b in range(30, -1, -1):
        eq = (st >> (b + 1)) == (prefix >> (b + 1))
        bit0 = (st & (1 << b)) == 0
        c0 = jnp.sum((eq & bit0).astype(jnp.int32), axis=0, keepdims=True)
        cond = kk <= c0
        prefix = jnp.where(cond, prefix, prefix | (1 << b))
        kk = jnp.where(cond, kk, kk - c0)
    return prefix


def _dist_deg_body(xb_ref, xf_ref, st_ref, dv_ref):
    xb = xb_ref[0]
    xf = xf_ref[0]
    xxt = jax.lax.dot_general(xf, xb, (((1,), (1,)), ((), ())),
                              preferred_element_type=jnp.float32)
    xsqp = jax.lax.dot_general(jnp.ones((1, _C), jnp.float32), xb * xb,
                               (((1,), (1,)), ((), ())),
                               preferred_element_type=jnp.float32)
    xsqn = jnp.sum(xf * xf, axis=1, keepdims=True)
    D = xsqp + (-2.0 * xxt) + xsqn
    bits = jax.lax.bitcast_convert_type(D, jnp.int32)
    st = jnp.where(bits < 0, bits ^ jnp.int32(0x7FFFFFFF), bits)
    st_ref[0] = st

    kvec = jnp.full((1, _BR), _K1, jnp.int32)
    v = _threshold_extract(st, _K1)
    m1 = _members_from_threshold(st, v, kvec).astype(jnp.float32)
    part = jnp.sum(m1, axis=1, keepdims=True)

    @pl.when(pl.program_id(1) == 0)
    def _init():
        dv_ref[0] = part

    @pl.when(pl.program_id(1) != 0)
    def _acc():
        dv_ref[0] += part


def _inc_s_body(st_ref, dvr_ref, S_ref, dvt_ref):
    kvec = dvr_ref[0].astype(jnp.int32)
    st = st_ref[0]
    v = _threshold_radix(st, kvec)
    m2 = _members_from_threshold(st, v, kvec).astype(jnp.float32)
    ri = jax.lax.broadcasted_iota(jnp.int32, (_N, _BR), 0)
    ci = jax.lax.broadcasted_iota(jnp.int32, (_N, _BR), 1) \
        + pl.program_id(1) * _BR
    nh = jnp.where(ri == ci, 1.0, m2)
    de = jnp.sum(nh, axis=0, keepdims=True)
    aw = nh * (1.0 / de)
    spart = jax.lax.dot_general(aw, nh, (((1,), (1,)), ((), ())),
                                preferred_element_type=jnp.float32)
    dpart = jnp.sum(nh, axis=1, keepdims=True)

    @pl.when(pl.program_id(1) == 0)
    def _init():
        S_ref[0] = spart
        dvt_ref[0] = dpart

    @pl.when(pl.program_id(1) != 0)
    def _acc():
        S_ref[0] += spart
        dvt_ref[0] += dpart


def _h_body(x_ref, Wc_ref, bc_ref, h_ref):
    h = jax.lax.dot_general(x_ref[0], Wc_ref[...], (((1,), (1,)), ((), ())),
                            preferred_element_type=jnp.float32)
    h_ref[0] = h + bc_ref[...]


def _gp_body(S_ref, L_ref, dvtc_ref, dvtr_ref, clc_ref, clr_ref, h_ref,
             P_ref):
    sc = jax.lax.rsqrt(dvtc_ref[0] + clc_ref[...])
    sr = jax.lax.rsqrt(dvtr_ref[0] + clr_ref[...])
    G = (S_ref[0] + L_ref[...]) * sc * sr
    P_ref[0] = jax.lax.dot_general(G, h_ref[0], (((1,), (0,)), ((), ())),
                                   preferred_element_type=jnp.float32)


def _bn_body(P_ref, x_ref, gamma_ref, beta_ref, o_ref):
    P = jnp.reshape(P_ref[...], (_B * _N, _C))
    m = jnp.mean(P, axis=0, keepdims=True)
    d = P - m
    var = jnp.mean(d * d, axis=0, keepdims=True)
    hn = d / jnp.sqrt(var + _EPS) * gamma_ref[...] + beta_ref[...]
    hr = jnp.maximum(hn, 0.0)
    o_ref[...] = jnp.reshape(hr, (_B, _N, _C)) + x_ref[...]


@jax.jit
def kernel(x, Wc, bc, gamma, beta):
    L = jnp.asarray(_LOC_L)
    clc = jnp.asarray(_LOC_CLOC).reshape(_N, 1)
    clr = jnp.asarray(_LOC_CLOC).reshape(1, _N)

    st, dv_col = pl.pallas_call(
        _dist_deg_body,
        grid=(_B, _NB),
        in_specs=[
            pl.BlockSpec((1, _BR, _C), lambda b, j: (b, j, 0)),
            pl.BlockSpec((1, _N, _C), lambda b, j: (b, 0, 0)),
        ],
        out_specs=[
            pl.BlockSpec((1, _N, _BR), lambda b, j: (b, 0, j)),
            pl.BlockSpec((1, _N, 1), lambda b, j: (b, 0, 0)),
        ],
        out_shape=[
            jax.ShapeDtypeStruct((_B, _N, _N), jnp.int32),
            jax.ShapeDtypeStruct((_B, _N, 1), jnp.float32),
        ],
    )(x, x)
    dv_row = jnp.swapaxes(dv_col, 1, 2)

    S, dvt_col = pl.pallas_call(
        _inc_s_body,
        grid=(_B, _NB),
        in_specs=[
            pl.BlockSpec((1, _N, _BR), lambda b, j: (b, 0, j)),
            pl.BlockSpec((1, 1, _BR), lambda b, j: (b, 0, j)),
        ],
        out_specs=[
            pl.BlockSpec((1, _N, _N), lambda b, j: (b, 0, 0)),
            pl.BlockSpec((1, _N, 1), lambda b, j: (b, 0, 0)),
        ],
        out_shape=[
            jax.ShapeDtypeStruct((_B, _N, _N), jnp.float32),
            jax.ShapeDtypeStruct((_B, _N, 1), jnp.float32),
        ],
    )(st, dv_row)
    dvt_row = jnp.swapaxes(dvt_col, 1, 2)

    h = pl.pallas_call(
        _h_body,
        grid=(_B,),
        in_specs=[
            pl.BlockSpec((1, _N, _C), lambda b: (b, 0, 0)),
            pl.BlockSpec((_C, _C), lambda b: (0, 0)),
            pl.BlockSpec((1, _C), lambda b: (0, 0)),
        ],
        out_specs=pl.BlockSpec((1, _N, _C), lambda b: (b, 0, 0)),
        out_shape=jax.ShapeDtypeStruct((_B, _N, _C), jnp.float32),
    )(x, Wc, bc.reshape(1, _C))

    P = pl.pallas_call(
        _gp_body,
        grid=(_B,),
        in_specs=[
            pl.BlockSpec((1, _N, _N), lambda b: (b, 0, 0)),
            pl.BlockSpec((_N, _N), lambda b: (0, 0)),
            pl.BlockSpec((1, _N, 1), lambda b: (b, 0, 0)),
            pl.BlockSpec((1, 1, _N), lambda b: (b, 0, 0)),
            pl.BlockSpec((_N, 1), lambda b: (0, 0)),
            pl.BlockSpec((1, _N), lambda b: (0, 0)),
            pl.BlockSpec((1, _N, _C), lambda b: (b, 0, 0)),
        ],
        out_specs=pl.BlockSpec((1, _N, _C), lambda b: (b, 0, 0)),
        out_shape=jax.ShapeDtypeStruct((_B, _N, _C), jnp.float32),
    )(S, L, dvt_col, dvt_row, clc, clr, h)

    out = pl.pallas_call(
        _bn_body,
        out_shape=jax.ShapeDtypeStruct((_B, _N, _C), jnp.float32),
    )(P, x, gamma.reshape(1, _C), beta.reshape(1, _C))
    return out


# speedup vs baseline: 52.1952x; 1.0423x over previous
import numpy as np
import jax
import jax.numpy as jnp
from jax.experimental import pallas as pl
from jax.experimental.pallas import tpu as pltpu

_B, _NODE, _C = 4, 32, 256
_N = _NODE * _NODE
_K1 = 11
_KS, _STRIDE = 5, 2
_EPS = 1e-5
_BR = 128
_NB = _N // _BR


def _local_parts():
    size, ks, stride = _NODE, _KS, _STRIDE
    inp = np.arange(size * size).reshape(size, size)
    patches = []
    for i in range(0, size - ks + 1, stride):
        for j in range(0, size - ks + 1, stride):
            patches.append(inp[i:i + ks, j:j + ks].reshape(-1))
    inp_unf = np.stack(patches, axis=0)
    edge = inp_unf.shape[0]
    H = np.zeros((size * size, edge), dtype=np.float32)
    for e in range(edge):
        H[inp_unf[e], e] = 1.0
    L = (H @ H.T) / float(ks * ks)
    cloc = H.sum(axis=1)
    return L.astype(np.float32), cloc.astype(np.float32)


_LOC_L, _LOC_CLOC = _local_parts()


def _members_from_threshold(st, v, kvec):
    n, p = st.shape
    lt = st < v
    c_lt = jnp.sum(lt.astype(jnp.int32), axis=0, keepdims=True)
    eqm = (st == v).astype(jnp.int32)
    csum = eqm
    d = 1
    while d < n:
        shifted = jnp.concatenate(
            [jnp.zeros((d, p), jnp.int32), csum[: n - d, :]], axis=0)
        csum = csum + shifted
        d *= 2
    excl = csum - eqm
    tie_take = eqm.astype(jnp.bool_) & (excl < (kvec - c_lt))
    return lt | tie_take


def _threshold_extract(st, k):
    rem = st
    removed = jnp.zeros((1, st.shape[1]), jnp.int32)
    v = jnp.full((1, st.shape[1]), -2147483648, jnp.int32)
    for _ in range(k):
        cur = jnp.min(rem, axis=0, keepdims=True)
        isv = rem == cur
        ccur = jnp.sum(isv.astype(jnp.int32), axis=0, keepdims=True)
        upd = removed < k
        v = jnp.where(upd, cur, v)
        removed = removed + jnp.where(upd, ccur, 0)
        rem = jnp.where(isv, jnp.int32(2147483647), rem)
    return v


def _threshold_radix(st, kvec):
    neg = (st < 0).astype(jnp.int32)
    c0 = jnp.sum(neg, axis=0, keepdims=True)
    cond = kvec <= c0
    prefix = jnp.where(cond, jnp.int32(-2147483648), jnp.int32(0))
    kk = jnp.where(cond, kvec, kvec - c0)
    for b in range(30, -1, -1):
        eq = (st >> (b + 1)) == (prefix >> (b + 1))
        bit0 = (st & (1 << b)) == 0
        c0 = jnp.sum((eq & bit0).astype(jnp.int32), axis=0, keepdims=True)
        cond = kk <= c0
        prefix = jnp.where(cond, prefix, prefix | (1 << b))
        kk = jnp.where(cond, kk, kk - c0)
    return prefix


def _dist_deg_body(xb_ref, xf_ref, st_ref, dv_ref):
    xb = xb_ref[0]
    xf = xf_ref[0]
    xxt = jax.lax.dot_general(xf, xb, (((1,), (1,)), ((), ())),
                              preferred_element_type=jnp.float32)
    xsqp = jax.lax.dot_general(jnp.ones((1, _C), jnp.float32), xb * xb,
                               (((1,), (1,)), ((), ())),
                               preferred_element_type=jnp.float32)
    xsqn = jnp.sum(xf * xf, axis=1, keepdims=True)
    D = xsqp + (-2.0 * xxt) + xsqn
    bits = jax.lax.bitcast_convert_type(D, jnp.int32)
    st = jnp.where(bits < 0, bits ^ jnp.int32(0x7FFFFFFF), bits)
    st_ref[0] = st

    kvec = jnp.full((1, _BR), _K1, jnp.int32)
    v = _threshold_extract(st, _K1)
    m1 = _members_from_threshold(st, v, kvec).astype(jnp.float32)
    part = jnp.sum(m1, axis=1, keepdims=True)

    @pl.when(pl.program_id(1) == 0)
    def _init():
        dv_ref[0] = part

    @pl.when(pl.program_id(1) != 0)
    def _acc():
        dv_ref[0] += part


def _inc_gp_body(st_ref, dvr_ref, x_ref, Wc_ref, bc_ref, L_ref, clc_ref,
                 clr_ref, P_ref, S_s, h_s, dvc_s, dvr_s):
    j = pl.program_id(1)

    @pl.when(j == 0)
    def _h():
        h = jax.lax.dot_general(x_ref[0], Wc_ref[...],
                                (((1,), (1,)), ((), ())),
                                preferred_element_type=jnp.float32)
        h_s[...] = h + bc_ref[...]

    kvec = dvr_ref[0].astype(jnp.int32)
    st = st_ref[0]
    v = _threshold_radix(st, kvec)
    m2 = _members_from_threshold(st, v, kvec).astype(jnp.float32)
    ri = jax.lax.broadcasted_iota(jnp.int32, (_N, _BR), 0)
    ci = jax.lax.broadcasted_iota(jnp.int32, (_N, _BR), 1) + j * _BR
    nh = jnp.where(ri == ci, 1.0, m2)
    de = jnp.sum(nh, axis=0, keepdims=True)
    aw = nh * (1.0 / de)
    spart = jax.lax.dot_general(aw, nh, (((1,), (1,)), ((), ())),
                                preferred_element_type=jnp.float32)
    dcol = jnp.sum(nh, axis=1, keepdims=True)
    drow = jax.lax.dot_general(jnp.ones((1, _BR), jnp.float32), nh,
                               (((1,), (1,)), ((), ())),
                               preferred_element_type=jnp.float32)

    @pl.when(j == 0)
    def _init():
        S_s[...] = spart
        dvc_s[...] = dcol
        dvr_s[...] = drow

    @pl.when(j != 0)
    def _acc():
        S_s[...] += spart
        dvc_s[...] += dcol
        dvr_s[...] += drow

    @pl.when(j == _NB - 1)
    def _fin():
        sc = jax.lax.rsqrt(dvc_s[...] + clc_ref[...])
        sr = jax.lax.rsqrt(dvr_s[...] + clr_ref[...])
        G = (S_s[...] + L_ref[...]) * sc * sr
        P_ref[0] = jax.lax.dot_general(G, h_s[...], (((1,), (0,)), ((), ())),
                                       preferred_element_type=jnp.float32)


def _bn_body(P_ref, x_ref, gamma_ref, beta_ref, o_ref):
    P = jnp.reshape(P_ref[...], (_B * _N, _C))
    m = jnp.mean(P, axis=0, keepdims=True)
    d = P - m
    var = jnp.mean(d * d, axis=0, keepdims=True)
    hn = d / jnp.sqrt(var + _EPS) * gamma_ref[...] + beta_ref[...]
    hr = jnp.maximum(hn, 0.0)
    o_ref[...] = jnp.reshape(hr, (_B, _N, _C)) + x_ref[...]


@jax.jit
def kernel(x, Wc, bc, gamma, beta):
    L = jnp.asarray(_LOC_L)
    clc = jnp.asarray(_LOC_CLOC).reshape(_N, 1)
    clr = jnp.asarray(_LOC_CLOC).reshape(1, _N)

    st, dv_col = pl.pallas_call(
        _dist_deg_body,
        grid=(_B, _NB),
        in_specs=[
            pl.BlockSpec((1, _BR, _C), lambda b, j: (b, j, 0)),
            pl.BlockSpec((1, _N, _C), lambda b, j: (b, 0, 0)),
        ],
        out_specs=[
            pl.BlockSpec((1, _N, _BR), lambda b, j: (b, 0, j)),
            pl.BlockSpec((1, _N, 1), lambda b, j: (b, 0, 0)),
        ],
        out_shape=[
            jax.ShapeDtypeStruct((_B, _N, _N), jnp.int32),
            jax.ShapeDtypeStruct((_B, _N, 1), jnp.float32),
        ],
    )(x, x)
    dv_row = jnp.swapaxes(dv_col, 1, 2)

    P = pl.pallas_call(
        _inc_gp_body,
        grid=(_B, _NB),
        in_specs=[
            pl.BlockSpec((1, _N, _BR), lambda b, j: (b, 0, j)),
            pl.BlockSpec((1, 1, _BR), lambda b, j: (b, 0, j)),
            pl.BlockSpec((1, _N, _C), lambda b, j: (b, 0, 0)),
            pl.BlockSpec((_C, _C), lambda b, j: (0, 0)),
            pl.BlockSpec((1, _C), lambda b, j: (0, 0)),
            pl.BlockSpec((_N, _N), lambda b, j: (0, 0)),
            pl.BlockSpec((_N, 1), lambda b, j: (0, 0)),
            pl.BlockSpec((1, _N), lambda b, j: (0, 0)),
        ],
        out_specs=pl.BlockSpec((1, _N, _C), lambda b, j: (b, 0, 0)),
        out_shape=jax.ShapeDtypeStruct((_B, _N, _C), jnp.float32),
        scratch_shapes=[
            pltpu.VMEM((_N, _N), jnp.float32),
            pltpu.VMEM((_N, _C), jnp.float32),
            pltpu.VMEM((_N, 1), jnp.float32),
            pltpu.VMEM((1, _N), jnp.float32),
        ],
    )(st, dv_row, x, Wc, bc.reshape(1, _C), L, clc, clr)

    out = pl.pallas_call(
        _bn_body,
        out_shape=jax.ShapeDtypeStruct((_B, _N, _C), jnp.float32),
    )(P, x, gamma.reshape(1, _C), beta.reshape(1, _C))
    return out
